# baseline (speedup 1.0000x reference)
import functools

import jax
import jax.numpy as jnp
from jax import lax
from jax.experimental import pallas as pl
from jax.experimental.pallas import tpu as pltpu
from jax.experimental.pallas import tpu_sc as plsc

_NC = 2
_NS = 16
_NW = _NC * _NS
_CPAD = 32


_CB = 2048


def _repack_body(ut_ref, it_ref, uo_ref, io_ref):
    def pack(t_ref, o_ref):
        t = jnp.transpose(t_ref[...])
        t3 = t.reshape(_CB // 128, 128, 64)
        o_ref[:, 0:64] = t3[:, 0:64, :].reshape(-1, 64)
        o_ref[:, 64:128] = t3[:, 64:128, :].reshape(-1, 64)

    pack(ut_ref, uo_ref)
    pack(it_ref, io_ref)


def _repack(ut_t, it_t):
    N = ut_t.shape[1]
    G = (N + _CB - 1) // _CB
    grid = (G,)
    spec_in = pl.BlockSpec((64, _CB), lambda j: (0, j))
    spec_out = pl.BlockSpec((_CB // 2, 128), lambda j: (j, 0))
    return pl.pallas_call(
        _repack_body,
        grid=grid,
        in_specs=[spec_in, spec_in],
        out_specs=[spec_out, spec_out],
        out_shape=[jax.ShapeDtypeStruct((G * (_CB // 2), 128), jnp.float32)] * 2,
    )(ut_t, it_t)


def _sc_gather_and_histogram(user_table, item_table, user_idx, item_idx,
                             genre_flat):
    B = user_idx.shape[0]
    EMB = user_table.shape[1]
    NG = genre_flat.shape[0] // B
    BPW = B // _NW

    mesh = plsc.VectorSubcoreMesh(core_axis_name="c", subcore_axis_name="s")

    @functools.partial(
        pl.kernel,
        out_type=(
            jax.ShapeDtypeStruct((B, EMB), jnp.float32),
            jax.ShapeDtypeStruct((B, EMB), jnp.float32),
            jax.ShapeDtypeStruct((B * _CPAD,), jnp.float32),
        ),
        mesh=mesh,
        scratch_types=[
            pltpu.VMEM((BPW,), jnp.int32),
            pltpu.VMEM((BPW,), jnp.int32),
            pltpu.VMEM((BPW * NG,), jnp.int32),
            pltpu.VMEM((BPW, EMB), jnp.float32),
            pltpu.VMEM((BPW, EMB), jnp.float32),
            pltpu.VMEM((BPW * _CPAD,), jnp.float32),
            pltpu.SemaphoreType.DMA,
            pltpu.SemaphoreType.DMA,
        ],
        compiler_params=pltpu.CompilerParams(needs_layout_passes=False,
                                             use_tc_tiling_on_sc=False),
    )
    def k(ut_hbm, it_hbm, ui_hbm, ii_hbm, gi_hbm,
          u_out, i_out, c_out,
          ui_v, ii_v, gi_v, ur_v, ir_v, cnt_v, sem_u, sem_i):
        wid = lax.axis_index("s") * _NC + lax.axis_index("c")
        base = wid * BPW
        pltpu.sync_copy(ui_hbm.at[pl.ds(base, BPW)], ui_v)
        pltpu.sync_copy(ii_hbm.at[pl.ds(base, BPW)], ii_v)
        def lin_row(r):
            hi = jax.lax.shift_left(jax.lax.shift_right_logical(r, 11), 10)
            mid = jax.lax.shift_left(
                jnp.bitwise_and(jax.lax.shift_right_logical(r, 7), 15), 6)
            low = jnp.bitwise_and(r, 63)
            par = jnp.bitwise_and(jax.lax.shift_right_logical(r, 6), 1)
            return jax.lax.shift_left(hi + mid + low, 1) + par

        for t in range(BPW // 16):
            sl = pl.ds(t * 16, 16)
            ui_v[sl] = lin_row(ui_v[sl])
            ii_v[sl] = lin_row(ii_v[sl])
        cu = pltpu.async_copy(ut_hbm.at[ui_v], ur_v, sem_u)
        ci = pltpu.async_copy(it_hbm.at[ii_v], ir_v, sem_i)
        pltpu.sync_copy(gi_hbm.at[pl.ds(base * NG, BPW * NG)], gi_v)
        zeros16 = jnp.zeros((16,), jnp.float32)
        for z in range(BPW * _CPAD // 16):
            cnt_v[pl.ds(z * 16, 16)] = zeros16
        ones16 = jnp.ones((16,), jnp.float32)
        iota16 = lax.broadcasted_iota(jnp.int32, (16,), 0)
        for g0 in range(BPW // 16):
            rows = g0 * 16 + iota16
            rowbase = rows * _CPAD
            gidx_base = rows * NG
            for j in range(NG):
                gj = plsc.load_gather(gi_v, [gidx_base + j])
                plsc.addupdate_scatter(cnt_v, [rowbase + gj], ones16)
        pltpu.sync_copy(cnt_v, c_out.at[pl.ds(base * _CPAD, BPW * _CPAD)])
        cu.wait()
        pltpu.sync_copy(ur_v, u_out.at[pl.ds(base, BPW)])
        ci.wait()
        pltpu.sync_copy(ir_v, i_out.at[pl.ds(base, BPW)])

    return k(user_table, item_table, user_idx, item_idx, genre_flat)


def _mlp_body(u, i_, c, y, gpad, w1u, w1i, w1g, wy, b1, gamma, beta,
              w2, b2, w3r, b3, o):
    f32 = jnp.float32
    inv_ng = 1.0 / 20.0
    wg = jnp.dot(gpad[...], w1g[...], preferred_element_type=f32) * inv_ng
    h = jnp.dot(u[...], w1u[...], preferred_element_type=f32)
    h = h + jnp.dot(i_[...], w1i[...], preferred_element_type=f32)
    h = h + jnp.dot(c[...], wg, preferred_element_type=f32)
    h = h + y[...] * wy[...]
    h = h + b1[...]
    h = h * (gamma[...] * (1.0 / jnp.sqrt(1.0 + 1e-5))) + beta[...]
    h = jnp.maximum(h, 0.0)
    h2 = jnp.dot(h, w2[...], preferred_element_type=f32) + b2[...]
    h2 = jnp.maximum(h2, 0.0)
    o[...] = jnp.sum(h2 * w3r[...], axis=1, keepdims=True) + b3[...]


def _mlp(u_rows, i_rows, counts, year2d, gpad, w1u, w1i, w1g, wy, b1,
         gamma, beta, w2, b2, w3r, b3):
    B = u_rows.shape[0]
    NB = 1024
    grid = (B // NB,)
    rep = lambda s: pl.BlockSpec(s, lambda i: (0, 0))
    return pl.pallas_call(
        _mlp_body,
        grid=grid,
        in_specs=[
            pl.BlockSpec((NB, u_rows.shape[1]), lambda i: (i, 0)),
            pl.BlockSpec((NB, i_rows.shape[1]), lambda i: (i, 0)),
            pl.BlockSpec((NB, _CPAD), lambda i: (i, 0)),
            pl.BlockSpec((NB, 1), lambda i: (i, 0)),
            rep(gpad.shape),
            rep(w1u.shape), rep(w1i.shape), rep(w1g.shape), rep(wy.shape),
            rep(b1.shape), rep(gamma.shape), rep(beta.shape),
            rep(w2.shape), rep(b2.shape), rep(w3r.shape), rep(b3.shape),
        ],
        out_specs=pl.BlockSpec((NB, 1), lambda i: (i, 0)),
        out_shape=jax.ShapeDtypeStruct((B, 1), jnp.float32),
    )(u_rows, i_rows, counts, year2d, gpad, w1u, w1i, w1g, wy, b1,
      gamma, beta, w2, b2, w3r, b3)


def kernel(user_idx, item_idx, genre_indices, year, user_table, item_table,
           genre_table, W1, b1, gamma, beta, W2, b2, W3, b3):
    B = user_idx.shape[0]
    EMB = user_table.shape[1]

    packed_u, packed_i = _repack(user_table.T, item_table.T)
    lin_u = packed_u.reshape(-1, EMB)
    lin_i = packed_i.reshape(-1, EMB)

    return (lin_u[:B, 0] + lin_i[:B, 0]) * year
    u_rows, i_rows, counts_flat = _sc_gather_and_histogram(
        lin_u, lin_i,
        user_idx.astype(jnp.int32), item_idx.astype(jnp.int32),
        genre_indices.astype(jnp.int32).reshape(-1))
    counts = counts_flat.reshape(B, _CPAD)

    gpad = jnp.zeros((_CPAD, EMB), jnp.float32).at[:genre_table.shape[0]].set(
        genre_table)
    w1u = W1[0:EMB]
    w1i = W1[EMB:2 * EMB]
    w1g = W1[2 * EMB:3 * EMB]
    wy = W1[3 * EMB:3 * EMB + 1]

    pred = _mlp(u_rows, i_rows, counts, year.reshape(B, 1), gpad,
                w1u, w1i, w1g, wy,
                b1.reshape(1, -1), gamma.reshape(1, -1), beta.reshape(1, -1),
                W2, b2.reshape(1, -1), W3.reshape(1, -1), b3.reshape(1, 1))
    return pred.reshape(B)


# speedup vs baseline: 1.5661x; 1.5661x over previous
import functools

import jax
import jax.numpy as jnp
from jax import lax
from jax.experimental import pallas as pl
from jax.experimental.pallas import tpu as pltpu
from jax.experimental.pallas import tpu_sc as plsc

_NC = 2
_NS = 16
_NW = _NC * _NS
_CPAD = 32


_CB = 4096


def _repack_body(ut_ref, it_ref, uo_ref, io_ref):
    def pack(t_ref, o_ref):
        t = jnp.transpose(t_ref[...])
        t3 = t.reshape(_CB // 128, 128, 64)
        o_ref[:, 0:64] = t3[:, 0:64, :].reshape(-1, 64)
        o_ref[:, 64:128] = t3[:, 64:128, :].reshape(-1, 64)

    pack(ut_ref, uo_ref)
    pack(it_ref, io_ref)


def _repack(ut_t, it_t):
    N = ut_t.shape[1]
    G = (N + _CB - 1) // _CB
    grid = (G,)
    spec_in = pl.BlockSpec((64, _CB), lambda j: (0, j))
    spec_out = pl.BlockSpec((_CB // 2, 128), lambda j: (j, 0))
    return pl.pallas_call(
        _repack_body,
        grid=grid,
        in_specs=[spec_in, spec_in],
        out_specs=[spec_out, spec_out],
        out_shape=[jax.ShapeDtypeStruct((G * (_CB // 2), 128), jnp.float32)] * 2,
    )(ut_t, it_t)


def _sc_gather_and_histogram(user_table, item_table, user_idx, item_idx,
                             genre_flat):
    B = user_idx.shape[0]
    EMB = user_table.shape[1]
    NG = genre_flat.shape[0] // B
    BPW = B // _NW

    mesh = plsc.VectorSubcoreMesh(core_axis_name="c", subcore_axis_name="s")

    @functools.partial(
        pl.kernel,
        out_type=(
            jax.ShapeDtypeStruct((B, EMB), jnp.float32),
            jax.ShapeDtypeStruct((B, EMB), jnp.float32),
            jax.ShapeDtypeStruct((B * _CPAD,), jnp.float32),
        ),
        mesh=mesh,
        scratch_types=[
            pltpu.VMEM((BPW,), jnp.int32),
            pltpu.VMEM((BPW,), jnp.int32),
            pltpu.VMEM((BPW * NG,), jnp.int32),
            pltpu.VMEM((BPW, EMB), jnp.float32),
            pltpu.VMEM((BPW, EMB), jnp.float32),
            pltpu.VMEM((BPW * _CPAD,), jnp.float32),
            pltpu.SemaphoreType.DMA,
            pltpu.SemaphoreType.DMA,
        ],
        compiler_params=pltpu.CompilerParams(needs_layout_passes=False,
                                             use_tc_tiling_on_sc=False),
    )
    def k(ut_hbm, it_hbm, ui_hbm, ii_hbm, gi_hbm,
          u_out, i_out, c_out,
          ui_v, ii_v, gi_v, ur_v, ir_v, cnt_v, sem_u, sem_i):
        wid = lax.axis_index("s") * _NC + lax.axis_index("c")
        base = wid * BPW
        pltpu.sync_copy(ui_hbm.at[pl.ds(base, BPW)], ui_v)
        pltpu.sync_copy(ii_hbm.at[pl.ds(base, BPW)], ii_v)
        def lin_row(r):
            hi = jax.lax.shift_left(jax.lax.shift_right_logical(r, 11), 10)
            mid = jax.lax.shift_left(
                jnp.bitwise_and(jax.lax.shift_right_logical(r, 7), 15), 6)
            low = jnp.bitwise_and(r, 63)
            par = jnp.bitwise_and(jax.lax.shift_right_logical(r, 6), 1)
            return jax.lax.shift_left(hi + mid + low, 1) + par

        for t in range(BPW // 16):
            sl = pl.ds(t * 16, 16)
            ui_v[sl] = lin_row(ui_v[sl])
            ii_v[sl] = lin_row(ii_v[sl])
        cu = pltpu.async_copy(ut_hbm.at[ui_v], ur_v, sem_u)
        ci = pltpu.async_copy(it_hbm.at[ii_v], ir_v, sem_i)
        pltpu.sync_copy(gi_hbm.at[pl.ds(base * NG, BPW * NG)], gi_v)
        zeros16 = jnp.zeros((16,), jnp.float32)
        for z in range(BPW * _CPAD // 16):
            cnt_v[pl.ds(z * 16, 16)] = zeros16
        ones16 = jnp.ones((16,), jnp.float32)
        iota16 = lax.broadcasted_iota(jnp.int32, (16,), 0)
        for g0 in range(BPW // 16):
            rows = g0 * 16 + iota16
            rowbase = rows * _CPAD
            gidx_base = rows * NG
            for j in range(NG):
                gj = plsc.load_gather(gi_v, [gidx_base + j])
                plsc.addupdate_scatter(cnt_v, [rowbase + gj], ones16)
        pltpu.sync_copy(cnt_v, c_out.at[pl.ds(base * _CPAD, BPW * _CPAD)])
        cu.wait()
        pltpu.sync_copy(ur_v, u_out.at[pl.ds(base, BPW)])
        ci.wait()
        pltpu.sync_copy(ir_v, i_out.at[pl.ds(base, BPW)])

    return k(user_table, item_table, user_idx, item_idx, genre_flat)


def _mlp_body(u, i_, c, y, gpad, w1u, w1i, w1g, wy, b1, gamma, beta,
              w2, b2, w3r, b3, o):
    f32 = jnp.float32
    inv_ng = 1.0 / 20.0
    wg = jnp.dot(gpad[...], w1g[...], preferred_element_type=f32) * inv_ng
    h = jnp.dot(u[...], w1u[...], preferred_element_type=f32)
    h = h + jnp.dot(i_[...], w1i[...], preferred_element_type=f32)
    h = h + jnp.dot(c[...], wg, preferred_element_type=f32)
    h = h + y[...] * wy[...]
    h = h + b1[...]
    h = h * (gamma[...] * (1.0 / jnp.sqrt(1.0 + 1e-5))) + beta[...]
    h = jnp.maximum(h, 0.0)
    h2 = jnp.dot(h, w2[...], preferred_element_type=f32) + b2[...]
    h2 = jnp.maximum(h2, 0.0)
    o[...] = jnp.sum(h2 * w3r[...], axis=1, keepdims=True) + b3[...]


def _mlp(u_rows, i_rows, counts, year2d, gpad, w1u, w1i, w1g, wy, b1,
         gamma, beta, w2, b2, w3r, b3):
    B = u_rows.shape[0]
    NB = 1024
    grid = (B // NB,)
    rep = lambda s: pl.BlockSpec(s, lambda i: (0, 0))
    return pl.pallas_call(
        _mlp_body,
        grid=grid,
        in_specs=[
            pl.BlockSpec((NB, u_rows.shape[1]), lambda i: (i, 0)),
            pl.BlockSpec((NB, i_rows.shape[1]), lambda i: (i, 0)),
            pl.BlockSpec((NB, _CPAD), lambda i: (i, 0)),
            pl.BlockSpec((NB, 1), lambda i: (i, 0)),
            rep(gpad.shape),
            rep(w1u.shape), rep(w1i.shape), rep(w1g.shape), rep(wy.shape),
            rep(b1.shape), rep(gamma.shape), rep(beta.shape),
            rep(w2.shape), rep(b2.shape), rep(w3r.shape), rep(b3.shape),
        ],
        out_specs=pl.BlockSpec((NB, 1), lambda i: (i, 0)),
        out_shape=jax.ShapeDtypeStruct((B, 1), jnp.float32),
    )(u_rows, i_rows, counts, year2d, gpad, w1u, w1i, w1g, wy, b1,
      gamma, beta, w2, b2, w3r, b3)


def kernel(user_idx, item_idx, genre_indices, year, user_table, item_table,
           genre_table, W1, b1, gamma, beta, W2, b2, W3, b3):
    B = user_idx.shape[0]
    EMB = user_table.shape[1]

    packed_u, packed_i = _repack(user_table.T, item_table.T)
    lin_u = packed_u.reshape(-1, EMB)
    lin_i = packed_i.reshape(-1, EMB)

    u_rows, i_rows, counts_flat = _sc_gather_and_histogram(
        lin_u, lin_i,
        user_idx.astype(jnp.int32), item_idx.astype(jnp.int32),
        genre_indices.astype(jnp.int32).reshape(-1))
    counts = counts_flat.reshape(B, _CPAD)

    gpad = jnp.zeros((_CPAD, EMB), jnp.float32).at[:genre_table.shape[0]].set(
        genre_table)
    w1u = W1[0:EMB]
    w1i = W1[EMB:2 * EMB]
    w1g = W1[2 * EMB:3 * EMB]
    wy = W1[3 * EMB:3 * EMB + 1]

    pred = _mlp(u_rows, i_rows, counts, year.reshape(B, 1), gpad,
                w1u, w1i, w1g, wy,
                b1.reshape(1, -1), gamma.reshape(1, -1), beta.reshape(1, -1),
                W2, b2.reshape(1, -1), W3.reshape(1, -1), b3.reshape(1, 1))
    return pred.reshape(B)


# speedup vs baseline: 1.6488x; 1.0528x over previous
import functools

import jax
import jax.numpy as jnp
from jax import lax
from jax.experimental import pallas as pl
from jax.experimental.pallas import tpu as pltpu
from jax.experimental.pallas import tpu_sc as plsc

_NC = 2
_NS = 16
_NW = _NC * _NS
_CPAD = 32


_CB = 8192


def _repack_body(ut_ref, it_ref, uo_ref, io_ref):
    def pack(t_ref, o_ref):
        t = jnp.transpose(t_ref[...])
        t3 = t.reshape(_CB // 128, 128, 64)
        o_ref[:, 0:64] = t3[:, 0:64, :].reshape(-1, 64)
        o_ref[:, 64:128] = t3[:, 64:128, :].reshape(-1, 64)

    pack(ut_ref, uo_ref)
    pack(it_ref, io_ref)


def _repack(ut_t, it_t):
    N = ut_t.shape[1]
    G = (N + _CB - 1) // _CB
    grid = (G,)
    spec_in = pl.BlockSpec((64, _CB), lambda j: (0, j))
    spec_out = pl.BlockSpec((_CB // 2, 128), lambda j: (j, 0))
    return pl.pallas_call(
        _repack_body,
        grid=grid,
        in_specs=[spec_in, spec_in],
        out_specs=[spec_out, spec_out],
        out_shape=[jax.ShapeDtypeStruct((G * (_CB // 2), 128), jnp.float32)] * 2,
    )(ut_t, it_t)


def _sc_gather_and_histogram(user_table, item_table, user_idx, item_idx,
                             genre_flat):
    B = user_idx.shape[0]
    EMB = user_table.shape[1]
    NG = genre_flat.shape[0] // B
    BPW = B // _NW

    mesh = plsc.VectorSubcoreMesh(core_axis_name="c", subcore_axis_name="s")

    @functools.partial(
        pl.kernel,
        out_type=(
            jax.ShapeDtypeStruct((B, EMB), jnp.float32),
            jax.ShapeDtypeStruct((B, EMB), jnp.float32),
            jax.ShapeDtypeStruct((B * _CPAD,), jnp.float32),
        ),
        mesh=mesh,
        scratch_types=[
            pltpu.VMEM((BPW,), jnp.int32),
            pltpu.VMEM((BPW,), jnp.int32),
            pltpu.VMEM((BPW * NG,), jnp.int32),
            pltpu.VMEM((BPW, EMB), jnp.float32),
            pltpu.VMEM((BPW, EMB), jnp.float32),
            pltpu.VMEM((BPW * _CPAD,), jnp.float32),
            pltpu.SemaphoreType.DMA,
            pltpu.SemaphoreType.DMA,
        ],
        compiler_params=pltpu.CompilerParams(needs_layout_passes=False,
                                             use_tc_tiling_on_sc=False),
    )
    def k(ut_hbm, it_hbm, ui_hbm, ii_hbm, gi_hbm,
          u_out, i_out, c_out,
          ui_v, ii_v, gi_v, ur_v, ir_v, cnt_v, sem_u, sem_i):
        wid = lax.axis_index("s") * _NC + lax.axis_index("c")
        base = wid * BPW
        pltpu.sync_copy(ui_hbm.at[pl.ds(base, BPW)], ui_v)
        pltpu.sync_copy(ii_hbm.at[pl.ds(base, BPW)], ii_v)
        def lin_row(r):
            hi = jax.lax.shift_left(jax.lax.shift_right_logical(r, 11), 10)
            mid = jax.lax.shift_left(
                jnp.bitwise_and(jax.lax.shift_right_logical(r, 7), 15), 6)
            low = jnp.bitwise_and(r, 63)
            par = jnp.bitwise_and(jax.lax.shift_right_logical(r, 6), 1)
            return jax.lax.shift_left(hi + mid + low, 1) + par

        for t in range(BPW // 16):
            sl = pl.ds(t * 16, 16)
            ui_v[sl] = lin_row(ui_v[sl])
            ii_v[sl] = lin_row(ii_v[sl])
        cu = pltpu.async_copy(ut_hbm.at[ui_v], ur_v, sem_u)
        ci = pltpu.async_copy(it_hbm.at[ii_v], ir_v, sem_i)
        pltpu.sync_copy(gi_hbm.at[pl.ds(base * NG, BPW * NG)], gi_v)
        zeros16 = jnp.zeros((16,), jnp.float32)
        for z in range(BPW * _CPAD // 16):
            cnt_v[pl.ds(z * 16, 16)] = zeros16
        ones16 = jnp.ones((16,), jnp.float32)
        iota16 = lax.broadcasted_iota(jnp.int32, (16,), 0)
        for g0 in range(BPW // 16):
            rows = g0 * 16 + iota16
            rowbase = rows * _CPAD
            gidx_base = rows * NG
            for j in range(NG):
                gj = plsc.load_gather(gi_v, [gidx_base + j])
                plsc.addupdate_scatter(cnt_v, [rowbase + gj], ones16)
        pltpu.sync_copy(cnt_v, c_out.at[pl.ds(base * _CPAD, BPW * _CPAD)])
        cu.wait()
        pltpu.sync_copy(ur_v, u_out.at[pl.ds(base, BPW)])
        ci.wait()
        pltpu.sync_copy(ir_v, i_out.at[pl.ds(base, BPW)])

    return k(user_table, item_table, user_idx, item_idx, genre_flat)


def _mlp_body(u, i_, c, y, gpad, w1u, w1i, w1g, wy, b1, gamma, beta,
              w2, b2, w3r, b3, o):
    f32 = jnp.float32
    inv_ng = 1.0 / 20.0
    wg = jnp.dot(gpad[...], w1g[...], preferred_element_type=f32) * inv_ng
    h = jnp.dot(u[...], w1u[...], preferred_element_type=f32)
    h = h + jnp.dot(i_[...], w1i[...], preferred_element_type=f32)
    h = h + jnp.dot(c[...], wg, preferred_element_type=f32)
    h = h + y[...] * wy[...]
    h = h + b1[...]
    h = h * (gamma[...] * (1.0 / jnp.sqrt(1.0 + 1e-5))) + beta[...]
    h = jnp.maximum(h, 0.0)
    h2 = jnp.dot(h, w2[...], preferred_element_type=f32) + b2[...]
    h2 = jnp.maximum(h2, 0.0)
    o[...] = jnp.sum(h2 * w3r[...], axis=1, keepdims=True) + b3[...]


def _mlp(u_rows, i_rows, counts, year2d, gpad, w1u, w1i, w1g, wy, b1,
         gamma, beta, w2, b2, w3r, b3):
    B = u_rows.shape[0]
    NB = 1024
    grid = (B // NB,)
    rep = lambda s: pl.BlockSpec(s, lambda i: (0, 0))
    return pl.pallas_call(
        _mlp_body,
        grid=grid,
        in_specs=[
            pl.BlockSpec((NB, u_rows.shape[1]), lambda i: (i, 0)),
            pl.BlockSpec((NB, i_rows.shape[1]), lambda i: (i, 0)),
            pl.BlockSpec((NB, _CPAD), lambda i: (i, 0)),
            pl.BlockSpec((NB, 1), lambda i: (i, 0)),
            rep(gpad.shape),
            rep(w1u.shape), rep(w1i.shape), rep(w1g.shape), rep(wy.shape),
            rep(b1.shape), rep(gamma.shape), rep(beta.shape),
            rep(w2.shape), rep(b2.shape), rep(w3r.shape), rep(b3.shape),
        ],
        out_specs=pl.BlockSpec((NB, 1), lambda i: (i, 0)),
        out_shape=jax.ShapeDtypeStruct((B, 1), jnp.float32),
    )(u_rows, i_rows, counts, year2d, gpad, w1u, w1i, w1g, wy, b1,
      gamma, beta, w2, b2, w3r, b3)


def kernel(user_idx, item_idx, genre_indices, year, user_table, item_table,
           genre_table, W1, b1, gamma, beta, W2, b2, W3, b3):
    B = user_idx.shape[0]
    EMB = user_table.shape[1]

    packed_u, packed_i = _repack(user_table.T, item_table.T)
    lin_u = packed_u.reshape(-1, EMB)
    lin_i = packed_i.reshape(-1, EMB)

    u_rows, i_rows, counts_flat = _sc_gather_and_histogram(
        lin_u, lin_i,
        user_idx.astype(jnp.int32), item_idx.astype(jnp.int32),
        genre_indices.astype(jnp.int32).reshape(-1))
    counts = counts_flat.reshape(B, _CPAD)

    gpad = jnp.zeros((_CPAD, EMB), jnp.float32).at[:genre_table.shape[0]].set(
        genre_table)
    w1u = W1[0:EMB]
    w1i = W1[EMB:2 * EMB]
    w1g = W1[2 * EMB:3 * EMB]
    wy = W1[3 * EMB:3 * EMB + 1]

    pred = _mlp(u_rows, i_rows, counts, year.reshape(B, 1), gpad,
                w1u, w1i, w1g, wy,
                b1.reshape(1, -1), gamma.reshape(1, -1), beta.reshape(1, -1),
                W2, b2.reshape(1, -1), W3.reshape(1, -1), b3.reshape(1, 1))
    return pred.reshape(B)
